# baseline (speedup 1.0000x reference)
import functools

import jax
import jax.numpy as jnp
from jax.experimental import pallas as pl
from jax.experimental.pallas import tpu as pltpu

CH = 200
NBUF = 4


def _body(x_ref, wt_ref, b_ref, adj_hbm, out_ref, buf, sem, *, n, d_out):
    nchunks = n // CH
    wt = wt_ref[...]
    bias = b_ref[...]

    def copy(c, slot):
        return pltpu.make_async_copy(
            adj_hbm.at[pl.ds(c * CH, CH), :],
            buf.at[slot],
            sem.at[slot],
        )

    for c in range(NBUF):
        copy(c, c).start()

    def loop(c, carry):
        slot = jax.lax.rem(c, NBUF)
        copy(c, slot).wait()
        a_bf = buf[slot].astype(jnp.bfloat16)
        h = jnp.dot(a_bf, x_ref[...], preferred_element_type=jnp.float32)
        out_ref[pl.ds(c * CH, CH), :] = (
            jnp.dot(h, wt, preferred_element_type=jnp.float32) + bias
        )

        @pl.when(c + NBUF < nchunks)
        def _next():
            copy(c + NBUF, slot).start()

        return carry

    jax.lax.fori_loop(0, nchunks, loop, 0)


def kernel(x, adj, W, b):
    n, d_in = x.shape
    d_out = W.shape[0]
    x_bf = x.astype(jnp.bfloat16)
    wt = W.T
    b2 = b.reshape(1, d_out)
    return pl.pallas_call(
        functools.partial(_body, n=n, d_out=d_out),
        in_specs=[
            pl.BlockSpec(memory_space=pltpu.MemorySpace.VMEM),
            pl.BlockSpec(memory_space=pltpu.MemorySpace.VMEM),
            pl.BlockSpec(memory_space=pltpu.MemorySpace.VMEM),
            pl.BlockSpec(memory_space=pltpu.MemorySpace.HBM),
        ],
        out_specs=pl.BlockSpec(memory_space=pltpu.MemorySpace.VMEM),
        out_shape=jax.ShapeDtypeStruct((n, d_out), jnp.float32),
        scratch_shapes=[
            pltpu.VMEM((NBUF, CH, n), jnp.float32),
            pltpu.SemaphoreType.DMA((NBUF,)),
        ],
        compiler_params=pltpu.CompilerParams(
            vmem_limit_bytes=64 * 1024 * 1024,
        ),
    )(x_bf, wt, b2, adj)
